# baseline (speedup 1.0000x reference)
import functools
import jax
import jax.numpy as jnp
from jax import lax
from jax.experimental import pallas as pl
from jax.experimental.pallas import tpu as pltpu
from jax.experimental.pallas import tpu_sc as plsc

_B, _N, _DIN, _H, _DOUT, _K = 4, 10000, 5, 128, 3, 16
_TR = 128
_TRK = 256
_NPAD = ((_N + _TRK - 1) // _TRK) * _TRK
_NT = _NPAD // _TR
_NTK = _NPAD // _TRK

_pcall = functools.partial(
    pl.pallas_call,
    compiler_params=pltpu.CompilerParams(
        dimension_semantics=("arbitrary",)))
_bf16 = jnp.bfloat16


def _bdot(a, b):
    return jnp.dot(a.astype(_bf16), b.astype(_bf16),
                   preferred_element_type=jnp.float32)


def _bf(x):
    return x.astype(_bf16).astype(jnp.float32)


def _full(shape):
    return pl.BlockSpec(shape, lambda i: (0,) * len(shape))


def _rows(d):
    return pl.BlockSpec((_TR, d), lambda i: (i, 0))


def _erows(d):
    return pl.BlockSpec((_TR * _K, d), lambda i: (i, 0))


def _enc_body(x_ref, w1, b1, w2, b2, w3, b3, ws, bs, lat_ref, skip_ref):
    x = x_ref[...]
    h = jnp.maximum(_bdot(x, w1[...]) + b1[...], 0.0)
    h = jnp.maximum(_bdot(h, w2[...]) + b2[...], 0.0)
    lat = _bdot(h, w3[...]) + b3[...]
    lat_ref[...] = lat
    skip_ref[...] = _bdot(lat, ws[...]) + bs[...]


def _encoder(xp, w1, b1, w2, b2, w3, b3, ws, bs):
    return _pcall(
        _enc_body,
        grid=(_NT,),
        in_specs=[_rows(8), _full((8, _H)), _full((1, _H)), _full((_H, _H)),
                  _full((1, _H)), _full((_H, 8)), _full((1, 8)),
                  _full((8, _H)), _full((1, _H))],
        out_specs=[_rows(8), _rows(_H)],
        out_shape=[jax.ShapeDtypeStruct((_NPAD, 8), jnp.float32),
                   jax.ShapeDtypeStruct((_NPAD, _H), jnp.float32)],
    )(xp, w1, b1, w2, b2, w3, b3, ws, bs)


def _knn_body(lat_ref, posT_ref, px_ref, py_ref, nbr_ref):
    i = pl.program_id(0)
    mask8 = (jax.lax.broadcasted_iota(jnp.int32, (1, 8), 1) < 2).astype(jnp.float32)
    posq = lat_ref[...] * mask8
    xr = lat_ref[:, 0:1]
    yr = lat_ref[:, 1:2]
    px = px_ref[...]
    py = py_ref[...]
    sq_r = xr * xr + yr * yr
    sq_c = px * px + py * py
    ip = _bdot(posq, posT_ref[...])
    d2 = (sq_r + sq_c) - 2.0 * ip
    col = jax.lax.broadcasted_iota(jnp.int32, (1, _NPAD), 1)
    row = i * _TRK + jax.lax.broadcasted_iota(jnp.int32, (_TRK, 1), 0)
    bad = (col >= _N) | (col == row)
    d2 = jnp.where(bad, jnp.inf, d2)
    for k in range(_K):
        mn = jnp.min(d2, axis=1, keepdims=True)
        idx = jnp.min(jnp.where(d2 == mn, col, _NPAD), axis=1, keepdims=True)
        nbr_ref[:, k:k + 1] = idx
        d2 = jnp.where(col == idx, jnp.inf, d2)


def _knn(lat, posT, px_row, py_row):
    return _pcall(
        _knn_body,
        grid=(_NTK,),
        in_specs=[pl.BlockSpec((_TRK, 8), lambda i: (i, 0)),
                  pl.BlockSpec((8, _NPAD), lambda i: (0, 0)),
                  pl.BlockSpec((1, _NPAD), lambda i: (0, 0)),
                  pl.BlockSpec((1, _NPAD), lambda i: (0, 0))],
        out_specs=pl.BlockSpec((_TRK, _K), lambda i: (i, 0)),
        out_shape=jax.ShapeDtypeStruct((_NPAD, _K), jnp.int32),
    )(lat, posT, px_row, py_row)


def _softmax_k(e):
    emax = jnp.max(e, axis=-1, keepdims=True)
    ee = jnp.exp(e - emax)
    den = jnp.sum(ee, axis=-1, keepdims=True) + 1e-16
    return ee / den


def _lrelu(x):
    return jnp.where(x >= 0, x, 0.2 * x)


def _att_e(m, att3):
    return jnp.sum(_bf(_lrelu(m)) * _bf(att3), axis=-1)


def _g1_body(latg_ref, lat_ref, wl, wr, we, att, b, wl2, wr2,
             xl2_ref, xr2_ref, ea_ref):
    g = latg_ref[...].reshape(_TR, _K, _H)
    ag = g[:, :, 2]
    s = lat_ref[:, 2:3]
    dx = g[:, :, 0] - lat_ref[:, 0:1]
    dy = g[:, :, 1] - lat_ref[:, 1:2]
    ea = jnp.sqrt(dx * dx + dy * dy)
    ea_ref[...] = ea
    wl3 = wl[...].reshape(1, 1, _H)
    wr3 = wr[...].reshape(1, 1, _H)
    we3 = we[...].reshape(1, 1, _H)
    att3 = att[...].reshape(1, 1, _H)
    m = ag[:, :, None] * wl3 + s[:, :, None] * wr3 + ea[:, :, None] * we3
    e = _att_e(m, att3)
    alpha = _softmax_k(e)
    t = jnp.sum(alpha * ag, axis=-1, keepdims=True)
    x1 = jnp.maximum(t * wl[...] + b[...], 0.0)
    xl2_ref[...] = _bdot(x1, wl2[...])
    xr2_ref[...] = _bdot(x1, wr2[...])


def _g1(latg, lat, wl, wr, we, att, b, wl2, wr2):
    return _pcall(
        _g1_body,
        grid=(_NT,),
        in_specs=[_erows(_H), _rows(8)] +
                 [_full((1, _H))] * 5 + [_full((_H, _H))] * 2,
        out_specs=[_rows(_H), _rows(_H), _rows(_K)],
        out_shape=[jax.ShapeDtypeStruct((_NPAD, _H), jnp.float32),
                   jax.ShapeDtypeStruct((_NPAD, _H), jnp.float32),
                   jax.ShapeDtypeStruct((_NPAD, _K), jnp.float32)],
    )(latg, lat, wl, wr, we, att, b, wl2, wr2)


def _g2_body(xg_ref, xr2_ref, ea_ref, we, att, b, wr3, wr4,
             x2_ref, xr3_ref, xr4_ref):
    xg = xg_ref[...].reshape(_TR, _K, _H)
    xr2 = xr2_ref[...]
    ea = ea_ref[...]
    we3 = we[...].reshape(1, 1, _H)
    att3 = att[...].reshape(1, 1, _H)
    m = xg + xr2[:, None, :] + ea[:, :, None] * we3
    e = _att_e(m, att3)
    alpha = _softmax_k(e)
    out = jnp.sum(alpha[:, :, None] * xg, axis=1)
    x2 = jnp.maximum(out + b[...], 0.0)
    x2_ref[...] = x2
    xr3_ref[...] = _bdot(x2, wr3[...])
    xr4_ref[...] = _bdot(x2, wr4[...])


def _g2(xg, xr2, ea, we, att, b, wr3, wr4):
    return _pcall(
        _g2_body,
        grid=(_NT,),
        in_specs=[_erows(_H), _rows(_H), _rows(_K),
                  _full((1, _H)), _full((1, _H)), _full((1, _H)),
                  _full((_H, _H)), _full((_H, _H))],
        out_specs=[_rows(_H), _rows(_H), _rows(_H)],
        out_shape=[jax.ShapeDtypeStruct((_NPAD, _H), jnp.float32)] * 3,
    )(xg, xr2, ea, we, att, b, wr3, wr4)


def _g34_body(xg_ref, xr3_ref, xr4_ref, skip_ref,
              wl3, att3, b3, wl4, att4, b4, lw, lb, vw, vb,
              lab_ref, val_ref):
    xg = xg_ref[...]
    skip = 0.1 * skip_ref[...]

    def branch(wl, att, xr, b):
        xlg = _bdot(xg, wl[...]).reshape(_TR, _K, _H)
        a3 = att[...].reshape(1, 1, _H)
        e = _att_e(xlg + xr[...][:, None, :], a3)
        alpha = _softmax_k(e)
        out = jnp.sum(alpha[:, :, None] * xlg, axis=1)
        return jnp.maximum(out + b[...] + skip, 0.0)

    x3 = branch(wl3, att3, xr3_ref, b3)
    lab_ref[...] = _bdot(x3, lw[...]) + lb[...]
    x4 = branch(wl4, att4, xr4_ref, b4)
    val_ref[...] = _bdot(x4, vw[...]) + vb[...]


def _g34(xg, xr3, xr4, skip, wl3, att3, b3, wl4, att4, b4, lw, lb, vw, vb):
    return _pcall(
        _g34_body,
        grid=(_NT,),
        in_specs=[_erows(_H), _rows(_H), _rows(_H), _rows(_H),
                  _full((_H, _H)), _full((1, _H)), _full((1, _H)),
                  _full((_H, _H)), _full((1, _H)), _full((1, _H)),
                  _full((_H, 8)), _full((1, 8)), _full((_H, 8)), _full((1, 8))],
        out_specs=[_rows(8), _rows(8)],
        out_shape=[jax.ShapeDtypeStruct((_NPAD, 8), jnp.float32),
                   jax.ShapeDtypeStruct((_NPAD, 8), jnp.float32)],
    )(xg, xr3, xr4, skip, wl3, att3, b3, wl4, att4, b4, lw, lb, vw, vb)


_NC, _NS = 2, 16
_NW = _NC * _NS
_TOTE = _NPAD * _K
_EPW = _TOTE // _NW
_RCH = 640
_NRCH = _EPW // _RCH

_mesh = functools.partial(plsc.VectorSubcoreMesh,
                          core_axis_name="c", subcore_axis_name="s")


def _wid():
    return lax.axis_index("s") * _NC + lax.axis_index("c")


def _sc_row_gather(table, idx):
    @functools.partial(
        pl.kernel, mesh=_mesh(),
        out_type=jax.ShapeDtypeStruct((_TOTE, _H), jnp.float32),
        scratch_types=[pltpu.VMEM((_RCH,), jnp.int32),
                       pltpu.VMEM((_RCH, _H), jnp.float32),
                       pltpu.SemaphoreType.DMA],
    )
    def k(table_hbm, idx_hbm, out_hbm, idxv, rows, sem):
        base = _wid() * _EPW

        def body(c, carry):
            off = base + c * _RCH
            pltpu.sync_copy(idx_hbm.at[pl.ds(off, _RCH)], idxv)
            pltpu.async_copy(table_hbm.at[idxv], rows, sem).wait()
            pltpu.sync_copy(rows, out_hbm.at[pl.ds(off, _RCH)])
            return carry

        lax.fori_loop(0, _NRCH, body, 0)

    return k(table, idx)


def kernel(batch, params):
    p = params
    f32 = jnp.float32
    w1 = jnp.zeros((8, _H), f32).at[:_DIN].set(p['enc_W1'])
    b1 = p['enc_b1'].reshape(1, _H)
    w2 = p['enc_W2']
    b2 = p['enc_b2'].reshape(1, _H)
    w3 = jnp.zeros((_H, 8), f32).at[:, :_DOUT].set(p['enc_W3'])
    b3 = jnp.zeros((1, 8), f32).at[0, :_DOUT].set(p['enc_b3'])
    ws = jnp.zeros((8, _H), f32).at[:_DOUT].set(p['skip_W'])
    bs = p['skip_b'].reshape(1, _H)
    lw = jnp.zeros((_H, 8), f32).at[:, :4].set(p['lab_W'])
    lb = jnp.zeros((1, 8), f32).at[0, :4].set(p['lab_b'])
    vw = jnp.zeros((_H, 8), f32).at[:, :1].set(p['val_W'])
    vb = jnp.zeros((1, 8), f32).at[0, :1].set(p['val_b'])

    labs, vals = [], []
    for b in range(_B):
        xp = jnp.zeros((_NPAD, 8), f32).at[:_N, :_DIN].set(batch[b])
        lat, skip = _encoder(xp, w1, b1, w2, b2, w3, b3, ws, bs)

        px = lat[:, 0]
        py = lat[:, 1]
        posT = jnp.zeros((8, _NPAD), f32).at[0].set(px).at[1].set(py)
        nbr = _knn(lat, posT, px.reshape(1, _NPAD), py.reshape(1, _NPAD))
        idx = nbr.reshape(-1)

        lat128 = jnp.zeros((_NPAD, _H), f32).at[:, :8].set(lat)
        latg = _sc_row_gather(lat128, idx)

        xl2, xr2, dist = _g1(latg, lat,
                             p['g1_Wl'], p['g1_Wr'], p['g1_We'],
                             p['g1_att'].reshape(1, _H),
                             p['g1_b'].reshape(1, _H),
                             p['g2_Wl'], p['g2_Wr'])

        xg2 = _sc_row_gather(xl2, idx)
        x2, xr3, xr4 = _g2(xg2, xr2, dist,
                           p['g2_We'], p['g2_att'].reshape(1, _H),
                           p['g2_b'].reshape(1, _H), p['g3_Wr'], p['g4_Wr'])

        xg = _sc_row_gather(x2, idx)
        lab, val = _g34(xg, xr3, xr4, skip,
                        p['g3_Wl'], p['g3_att'].reshape(1, _H),
                        p['g3_b'].reshape(1, _H),
                        p['g4_Wl'], p['g4_att'].reshape(1, _H),
                        p['g4_b'].reshape(1, _H),
                        lw, lb, vw, vb)
        labs.append(lab[:_N, :4])
        vals.append(val[:_N, :1])

    return jnp.stack(labs), jnp.stack(vals)


# speedup vs baseline: 1.2259x; 1.2259x over previous
import functools
import jax
import jax.numpy as jnp
from jax import lax
from jax.experimental import pallas as pl
from jax.experimental.pallas import tpu as pltpu
from jax.experimental.pallas import tpu_sc as plsc

_B, _N, _DIN, _H, _DOUT, _K = 4, 10000, 5, 128, 3, 16
_TR = 128
_NPAD = ((_N + _TR - 1) // _TR) * _TR
_NT = _NPAD // _TR

_pcall = functools.partial(
    pl.pallas_call,
    compiler_params=pltpu.CompilerParams(
        dimension_semantics=("arbitrary",)))
_bf16 = jnp.bfloat16


def _bdot(a, b):
    return jnp.dot(a.astype(_bf16), b.astype(_bf16),
                   preferred_element_type=jnp.float32)


def _bf(x):
    return x.astype(_bf16).astype(jnp.float32)


def _full(shape):
    return pl.BlockSpec(shape, lambda i: (0,) * len(shape))


def _rows(d):
    return pl.BlockSpec((_TR, d), lambda i: (i, 0))


def _erows(d):
    return pl.BlockSpec((_TR * _K, d), lambda i: (i, 0))


def _enc_body(x_ref, w1, b1, w2, b2, w3, b3, ws, bs, lat_ref, skip_ref):
    x = x_ref[...]
    h = jnp.maximum(_bdot(x, w1[...]) + b1[...], 0.0)
    h = jnp.maximum(_bdot(h, w2[...]) + b2[...], 0.0)
    lat = _bdot(h, w3[...]) + b3[...]
    lat_ref[...] = lat
    skip_ref[...] = _bdot(lat, ws[...]) + bs[...]


def _encoder(xp, w1, b1, w2, b2, w3, b3, ws, bs):
    return _pcall(
        _enc_body,
        grid=(_NT,),
        in_specs=[_rows(8), _full((8, _H)), _full((1, _H)), _full((_H, _H)),
                  _full((1, _H)), _full((_H, 8)), _full((1, 8)),
                  _full((8, _H)), _full((1, _H))],
        out_specs=[_rows(8), _rows(_H)],
        out_shape=[jax.ShapeDtypeStruct((_NPAD, 8), jnp.float32),
                   jax.ShapeDtypeStruct((_NPAD, _H), jnp.float32)],
    )(xp, w1, b1, w2, b2, w3, b3, ws, bs)


def _knn_body(lat_ref, posT_ref, px_ref, py_ref, nbr_ref):
    i = pl.program_id(0)
    mask8 = (jax.lax.broadcasted_iota(jnp.int32, (1, 8), 1) < 2).astype(jnp.float32)
    posq = lat_ref[...] * mask8
    xr = lat_ref[:, 0:1]
    yr = lat_ref[:, 1:2]
    px = px_ref[...]
    py = py_ref[...]
    sq_r = xr * xr + yr * yr
    sq_c = px * px + py * py
    ip = _bdot(posq, posT_ref[...])
    d2 = (sq_r + sq_c) - 2.0 * ip
    col = jax.lax.broadcasted_iota(jnp.int32, (1, _NPAD), 1)
    row = i * _TR + jax.lax.broadcasted_iota(jnp.int32, (_TR, 1), 0)
    bad = (col >= _N) | (col == row)
    d2 = jnp.where(bad, jnp.inf, d2)
    for k in range(_K):
        idx = jnp.argmin(d2, axis=1, keepdims=True).astype(jnp.int32)
        nbr_ref[:, k:k + 1] = idx
        d2 = jnp.where(col == idx, jnp.inf, d2)


def _knn(lat, posT, px_row, py_row):
    return _pcall(
        _knn_body,
        grid=(_NT,),
        in_specs=[_rows(8),
                  pl.BlockSpec((8, _NPAD), lambda i: (0, 0)),
                  pl.BlockSpec((1, _NPAD), lambda i: (0, 0)),
                  pl.BlockSpec((1, _NPAD), lambda i: (0, 0))],
        out_specs=_rows(_K),
        out_shape=jax.ShapeDtypeStruct((_NPAD, _K), jnp.int32),
    )(lat, posT, px_row, py_row)


def _softmax_k(e):
    emax = jnp.max(e, axis=-1, keepdims=True)
    ee = jnp.exp(e - emax)
    den = jnp.sum(ee, axis=-1, keepdims=True) + 1e-16
    return ee / den


def _lrelu(x):
    return jnp.where(x >= 0, x, 0.2 * x)


def _att_e(m, att3):
    return jnp.sum(_bf(_lrelu(m)) * _bf(att3), axis=-1)


def _g1_body(latg_ref, lat_ref, wl, wr, we, att, b, wl2, wr2,
             xl2_ref, xr2_ref, ea_ref):
    g = latg_ref[...].reshape(_TR, _K, _H)
    ag = g[:, :, 2]
    s = lat_ref[:, 2:3]
    dx = g[:, :, 0] - lat_ref[:, 0:1]
    dy = g[:, :, 1] - lat_ref[:, 1:2]
    ea = jnp.sqrt(dx * dx + dy * dy)
    ea_ref[...] = ea
    wl3 = wl[...].reshape(1, 1, _H)
    wr3 = wr[...].reshape(1, 1, _H)
    we3 = we[...].reshape(1, 1, _H)
    att3 = att[...].reshape(1, 1, _H)
    m = ag[:, :, None] * wl3 + s[:, :, None] * wr3 + ea[:, :, None] * we3
    e = _att_e(m, att3)
    alpha = _softmax_k(e)
    t = jnp.sum(alpha * ag, axis=-1, keepdims=True)
    x1 = jnp.maximum(t * wl[...] + b[...], 0.0)
    xl2_ref[...] = _bdot(x1, wl2[...])
    xr2_ref[...] = _bdot(x1, wr2[...])


def _g1(latg, lat, wl, wr, we, att, b, wl2, wr2):
    return _pcall(
        _g1_body,
        grid=(_NT,),
        in_specs=[_erows(_H), _rows(8)] +
                 [_full((1, _H))] * 5 + [_full((_H, _H))] * 2,
        out_specs=[_rows(_H), _rows(_H), _rows(_K)],
        out_shape=[jax.ShapeDtypeStruct((_NPAD, _H), jnp.float32),
                   jax.ShapeDtypeStruct((_NPAD, _H), jnp.float32),
                   jax.ShapeDtypeStruct((_NPAD, _K), jnp.float32)],
    )(latg, lat, wl, wr, we, att, b, wl2, wr2)


def _g2_body(xg_ref, xr2_ref, ea_ref, we, att, b, wr3, wr4,
             x2_ref, xr3_ref, xr4_ref):
    xg = xg_ref[...].reshape(_TR, _K, _H)
    xr2 = xr2_ref[...]
    ea = ea_ref[...]
    we3 = we[...].reshape(1, 1, _H)
    att3 = att[...].reshape(1, 1, _H)
    m = xg + xr2[:, None, :] + ea[:, :, None] * we3
    e = _att_e(m, att3)
    alpha = _softmax_k(e)
    out = jnp.sum(alpha[:, :, None] * xg, axis=1)
    x2 = jnp.maximum(out + b[...], 0.0)
    x2_ref[...] = x2
    xr3_ref[...] = _bdot(x2, wr3[...])
    xr4_ref[...] = _bdot(x2, wr4[...])


def _g2(xg, xr2, ea, we, att, b, wr3, wr4):
    return _pcall(
        _g2_body,
        grid=(_NT,),
        in_specs=[_erows(_H), _rows(_H), _rows(_K),
                  _full((1, _H)), _full((1, _H)), _full((1, _H)),
                  _full((_H, _H)), _full((_H, _H))],
        out_specs=[_rows(_H), _rows(_H), _rows(_H)],
        out_shape=[jax.ShapeDtypeStruct((_NPAD, _H), jnp.float32)] * 3,
    )(xg, xr2, ea, we, att, b, wr3, wr4)


def _g34_body(xg_ref, xr3_ref, xr4_ref, skip_ref,
              wl3, att3, b3, wl4, att4, b4, lw, lb, vw, vb,
              lab_ref, val_ref):
    xg = xg_ref[...]
    skip = 0.1 * skip_ref[...]

    def branch(wl, att, xr, b):
        xlg = _bdot(xg, wl[...]).reshape(_TR, _K, _H)
        a3 = att[...].reshape(1, 1, _H)
        e = _att_e(xlg + xr[...][:, None, :], a3)
        alpha = _softmax_k(e)
        out = jnp.sum(alpha[:, :, None] * xlg, axis=1)
        return jnp.maximum(out + b[...] + skip, 0.0)

    x3 = branch(wl3, att3, xr3_ref, b3)
    lab_ref[...] = _bdot(x3, lw[...]) + lb[...]
    x4 = branch(wl4, att4, xr4_ref, b4)
    val_ref[...] = _bdot(x4, vw[...]) + vb[...]


def _g34(xg, xr3, xr4, skip, wl3, att3, b3, wl4, att4, b4, lw, lb, vw, vb):
    return _pcall(
        _g34_body,
        grid=(_NT,),
        in_specs=[_erows(_H), _rows(_H), _rows(_H), _rows(_H),
                  _full((_H, _H)), _full((1, _H)), _full((1, _H)),
                  _full((_H, _H)), _full((1, _H)), _full((1, _H)),
                  _full((_H, 8)), _full((1, 8)), _full((_H, 8)), _full((1, 8))],
        out_specs=[_rows(8), _rows(8)],
        out_shape=[jax.ShapeDtypeStruct((_NPAD, 8), jnp.float32),
                   jax.ShapeDtypeStruct((_NPAD, 8), jnp.float32)],
    )(xg, xr3, xr4, skip, wl3, att3, b3, wl4, att4, b4, lw, lb, vw, vb)


_NC, _NS = 2, 16
_NW = _NC * _NS
_TOTE = _NPAD * _K
_EPW = _TOTE // _NW
_RCH = 632
_NRCH = _EPW // _RCH

_mesh = functools.partial(plsc.VectorSubcoreMesh,
                          core_axis_name="c", subcore_axis_name="s")


def _wid():
    return lax.axis_index("s") * _NC + lax.axis_index("c")


def _sc_row_gather(table, idx):
    @functools.partial(
        pl.kernel, mesh=_mesh(),
        out_type=jax.ShapeDtypeStruct((_TOTE, _H), jnp.float32),
        scratch_types=[pltpu.VMEM((_RCH,), jnp.int32),
                       pltpu.VMEM((_RCH, _H), jnp.float32),
                       pltpu.SemaphoreType.DMA],
    )
    def k(table_hbm, idx_hbm, out_hbm, idxv, rows, sem):
        base = _wid() * _EPW

        def body(c, carry):
            off = base + c * _RCH
            pltpu.sync_copy(idx_hbm.at[pl.ds(off, _RCH)], idxv)
            pltpu.async_copy(table_hbm.at[idxv], rows, sem).wait()
            pltpu.sync_copy(rows, out_hbm.at[pl.ds(off, _RCH)])
            return carry

        lax.fori_loop(0, _NRCH, body, 0)

    return k(table, idx)


def kernel(batch, params):
    p = params
    f32 = jnp.float32
    w1 = jnp.zeros((8, _H), f32).at[:_DIN].set(p['enc_W1'])
    b1 = p['enc_b1'].reshape(1, _H)
    w2 = p['enc_W2']
    b2 = p['enc_b2'].reshape(1, _H)
    w3 = jnp.zeros((_H, 8), f32).at[:, :_DOUT].set(p['enc_W3'])
    b3 = jnp.zeros((1, 8), f32).at[0, :_DOUT].set(p['enc_b3'])
    ws = jnp.zeros((8, _H), f32).at[:_DOUT].set(p['skip_W'])
    bs = p['skip_b'].reshape(1, _H)
    lw = jnp.zeros((_H, 8), f32).at[:, :4].set(p['lab_W'])
    lb = jnp.zeros((1, 8), f32).at[0, :4].set(p['lab_b'])
    vw = jnp.zeros((_H, 8), f32).at[:, :1].set(p['val_W'])
    vb = jnp.zeros((1, 8), f32).at[0, :1].set(p['val_b'])

    labs, vals = [], []
    for b in range(_B):
        xp = jnp.zeros((_NPAD, 8), f32).at[:_N, :_DIN].set(batch[b])
        lat, skip = _encoder(xp, w1, b1, w2, b2, w3, b3, ws, bs)

        px = lat[:, 0]
        py = lat[:, 1]
        posT = jnp.zeros((8, _NPAD), f32).at[0].set(px).at[1].set(py)
        nbr = _knn(lat, posT, px.reshape(1, _NPAD), py.reshape(1, _NPAD))
        idx = nbr.reshape(-1)

        lat128 = jnp.zeros((_NPAD, _H), f32).at[:, :8].set(lat)
        latg = _sc_row_gather(lat128, idx)

        xl2, xr2, dist = _g1(latg, lat,
                             p['g1_Wl'], p['g1_Wr'], p['g1_We'],
                             p['g1_att'].reshape(1, _H),
                             p['g1_b'].reshape(1, _H),
                             p['g2_Wl'], p['g2_Wr'])

        xg2 = _sc_row_gather(xl2, idx)
        x2, xr3, xr4 = _g2(xg2, xr2, dist,
                           p['g2_We'], p['g2_att'].reshape(1, _H),
                           p['g2_b'].reshape(1, _H), p['g3_Wr'], p['g4_Wr'])

        xg = _sc_row_gather(x2, idx)
        lab, val = _g34(xg, xr3, xr4, skip,
                        p['g3_Wl'], p['g3_att'].reshape(1, _H),
                        p['g3_b'].reshape(1, _H),
                        p['g4_Wl'], p['g4_att'].reshape(1, _H),
                        p['g4_b'].reshape(1, _H),
                        lw, lb, vw, vb)
        labs.append(lab[:_N, :4])
        vals.append(val[:_N, :1])

    return jnp.stack(labs), jnp.stack(vals)


# speedup vs baseline: 1.2387x; 1.0104x over previous
import functools
import jax
import jax.numpy as jnp
from jax import lax
from jax.experimental import pallas as pl
from jax.experimental.pallas import tpu as pltpu
from jax.experimental.pallas import tpu_sc as plsc

_B, _N, _DIN, _H, _DOUT, _K = 4, 10000, 5, 128, 3, 16
_TR = 128
_NPAD = ((_N + _TR - 1) // _TR) * _TR
_NT = _NPAD // _TR

_pcall = functools.partial(
    pl.pallas_call,
    compiler_params=pltpu.CompilerParams(
        dimension_semantics=("arbitrary",)))
_bf16 = jnp.bfloat16


def _bdot(a, b):
    return jnp.dot(a.astype(_bf16), b.astype(_bf16),
                   preferred_element_type=jnp.float32)


def _bf(x):
    return x.astype(_bf16).astype(jnp.float32)


def _full(shape):
    return pl.BlockSpec(shape, lambda i: (0,) * len(shape))


def _rows(d):
    return pl.BlockSpec((_TR, d), lambda i: (i, 0))


def _erows(d):
    return pl.BlockSpec((_TR * _K, d), lambda i: (i, 0))


def _enc_body(x_ref, w1, b1, w2, b2, w3, b3, ws, bs, lat_ref, skip_ref):
    x = x_ref[...]
    h = jnp.maximum(_bdot(x, w1[...]) + b1[...], 0.0)
    h = jnp.maximum(_bdot(h, w2[...]) + b2[...], 0.0)
    lat = _bdot(h, w3[...]) + b3[...]
    lat_ref[...] = lat
    skip_ref[...] = _bdot(lat, ws[...]) + bs[...]


def _encoder(xp, w1, b1, w2, b2, w3, b3, ws, bs):
    return _pcall(
        _enc_body,
        grid=(_NT,),
        in_specs=[_rows(8), _full((8, _H)), _full((1, _H)), _full((_H, _H)),
                  _full((1, _H)), _full((_H, 8)), _full((1, 8)),
                  _full((8, _H)), _full((1, _H))],
        out_specs=[_rows(8), _rows(_H)],
        out_shape=[jax.ShapeDtypeStruct((_NPAD, 8), jnp.float32),
                   jax.ShapeDtypeStruct((_NPAD, _H), jnp.float32)],
    )(xp, w1, b1, w2, b2, w3, b3, ws, bs)


def _knn_body(lat_ref, posT_ref, px_ref, py_ref, nbr_ref):
    i = pl.program_id(0)
    mask8 = (jax.lax.broadcasted_iota(jnp.int32, (1, 8), 1) < 2).astype(jnp.float32)
    posq = lat_ref[...] * mask8
    xr = lat_ref[:, 0:1]
    yr = lat_ref[:, 1:2]
    px = px_ref[...]
    py = py_ref[...]
    sq_r = xr * xr + yr * yr
    sq_c = px * px + py * py
    ip = _bdot(posq, posT_ref[...])
    d2 = (sq_r + sq_c) - 2.0 * ip
    col = jax.lax.broadcasted_iota(jnp.int32, (1, _NPAD), 1)
    row = i * _TR + jax.lax.broadcasted_iota(jnp.int32, (_TR, 1), 0)
    bad = (col >= _N) | (col == row)
    d2 = jnp.where(bad, jnp.inf, d2)
    for k in range(_K):
        idx = jnp.argmin(d2, axis=1, keepdims=True).astype(jnp.int32)
        nbr_ref[:, k:k + 1] = idx
        d2 = jnp.where(col == idx, jnp.inf, d2)


def _knn(lat, posT, px_row, py_row):
    return _pcall(
        _knn_body,
        grid=(_NT,),
        in_specs=[_rows(8),
                  pl.BlockSpec((8, _NPAD), lambda i: (0, 0)),
                  pl.BlockSpec((1, _NPAD), lambda i: (0, 0)),
                  pl.BlockSpec((1, _NPAD), lambda i: (0, 0))],
        out_specs=_rows(_K),
        out_shape=jax.ShapeDtypeStruct((_NPAD, _K), jnp.int32),
    )(lat, posT, px_row, py_row)


def _softmax_k(e):
    emax = jnp.max(e, axis=-1, keepdims=True)
    ee = jnp.exp(e - emax)
    den = jnp.sum(ee, axis=-1, keepdims=True) + 1e-16
    return ee / den


def _lrelu(x):
    return jnp.where(x >= 0, x, 0.2 * x)


def _att_e(m, att3):
    return jnp.sum(_bf(_lrelu(m)) * _bf(att3), axis=-1)


def _g1_body(latg_ref, lat_ref, wl, wr, we, att, b, wl2, wr2,
             xl2_ref, xr2_ref, ea_ref):
    g = latg_ref[...].reshape(_TR, _K, _H)
    ag = g[:, :, 2]
    s = lat_ref[:, 2:3]
    dx = g[:, :, 0] - lat_ref[:, 0:1]
    dy = g[:, :, 1] - lat_ref[:, 1:2]
    ea = jnp.sqrt(dx * dx + dy * dy)
    ea_ref[...] = ea
    wl3 = wl[...].reshape(1, 1, _H)
    wr3 = wr[...].reshape(1, 1, _H)
    we3 = we[...].reshape(1, 1, _H)
    att3 = att[...].reshape(1, 1, _H)
    m = ag[:, :, None] * wl3 + s[:, :, None] * wr3 + ea[:, :, None] * we3
    e = _att_e(m, att3)
    alpha = _softmax_k(e)
    t = jnp.sum(alpha * ag, axis=-1, keepdims=True)
    x1 = jnp.maximum(t * wl[...] + b[...], 0.0)
    xl2_ref[...] = _bdot(x1, wl2[...])
    xr2_ref[...] = _bdot(x1, wr2[...])


def _g1(latg, lat, wl, wr, we, att, b, wl2, wr2):
    return _pcall(
        _g1_body,
        grid=(_NT,),
        in_specs=[_erows(_H), _rows(8)] +
                 [_full((1, _H))] * 5 + [_full((_H, _H))] * 2,
        out_specs=[_rows(_H), _rows(_H), _rows(_K)],
        out_shape=[jax.ShapeDtypeStruct((_NPAD, _H), jnp.float32),
                   jax.ShapeDtypeStruct((_NPAD, _H), jnp.float32),
                   jax.ShapeDtypeStruct((_NPAD, _K), jnp.float32)],
    )(latg, lat, wl, wr, we, att, b, wl2, wr2)


def _g2_body(xg_ref, xr2_ref, ea_ref, we, att, b, wr3, wr4,
             x2_ref, xr3_ref, xr4_ref):
    xg = xg_ref[...].reshape(_TR, _K, _H)
    xr2 = xr2_ref[...]
    ea = ea_ref[...]
    we3 = we[...].reshape(1, 1, _H)
    att3 = att[...].reshape(1, 1, _H)
    m = xg + xr2[:, None, :] + ea[:, :, None] * we3
    e = _att_e(m, att3)
    alpha = _softmax_k(e)
    out = jnp.sum(alpha[:, :, None] * xg, axis=1)
    x2 = jnp.maximum(out + b[...], 0.0)
    x2_ref[...] = x2
    xr3_ref[...] = _bdot(x2, wr3[...])
    xr4_ref[...] = _bdot(x2, wr4[...])


def _g2(xg, xr2, ea, we, att, b, wr3, wr4):
    return _pcall(
        _g2_body,
        grid=(_NT,),
        in_specs=[_erows(_H), _rows(_H), _rows(_K),
                  _full((1, _H)), _full((1, _H)), _full((1, _H)),
                  _full((_H, _H)), _full((_H, _H))],
        out_specs=[_rows(_H), _rows(_H), _rows(_H)],
        out_shape=[jax.ShapeDtypeStruct((_NPAD, _H), jnp.float32)] * 3,
    )(xg, xr2, ea, we, att, b, wr3, wr4)


def _g34_body(xg_ref, xr3_ref, xr4_ref, skip_ref,
              wl3, att3, b3, wl4, att4, b4, lw, lb, vw, vb,
              lab_ref, val_ref):
    xg = xg_ref[...]
    skip = 0.1 * skip_ref[...]

    def branch(wl, att, xr, b):
        xlg = _bdot(xg, wl[...]).reshape(_TR, _K, _H)
        a3 = att[...].reshape(1, 1, _H)
        e = _att_e(xlg + xr[...][:, None, :], a3)
        alpha = _softmax_k(e)
        out = jnp.sum(alpha[:, :, None] * xlg, axis=1)
        return jnp.maximum(out + b[...] + skip, 0.0)

    x3 = branch(wl3, att3, xr3_ref, b3)
    lab_ref[...] = _bdot(x3, lw[...]) + lb[...]
    x4 = branch(wl4, att4, xr4_ref, b4)
    val_ref[...] = _bdot(x4, vw[...]) + vb[...]


def _g34(xg, xr3, xr4, skip, wl3, att3, b3, wl4, att4, b4, lw, lb, vw, vb):
    return _pcall(
        _g34_body,
        grid=(_NT,),
        in_specs=[_erows(_H), _rows(_H), _rows(_H), _rows(_H),
                  _full((_H, _H)), _full((1, _H)), _full((1, _H)),
                  _full((_H, _H)), _full((1, _H)), _full((1, _H)),
                  _full((_H, 8)), _full((1, 8)), _full((_H, 8)), _full((1, 8))],
        out_specs=[_rows(8), _rows(8)],
        out_shape=[jax.ShapeDtypeStruct((_NPAD, 8), jnp.float32),
                   jax.ShapeDtypeStruct((_NPAD, 8), jnp.float32)],
    )(xg, xr3, xr4, skip, wl3, att3, b3, wl4, att4, b4, lw, lb, vw, vb)


_NC, _NS = 2, 16
_NW = _NC * _NS
_TOTE = _NPAD * _K
_EPW = _TOTE // _NW
_RCH = 64
_NRCH = _EPW // _RCH

_mesh = functools.partial(plsc.VectorSubcoreMesh,
                          core_axis_name="c", subcore_axis_name="s")


def _wid():
    return lax.axis_index("s") * _NC + lax.axis_index("c")


def _sc_row_gather(table, idx):
    @functools.partial(
        pl.kernel, mesh=_mesh(),
        out_type=jax.ShapeDtypeStruct((_TOTE, _H), jnp.float32),
        scratch_types=[pltpu.VMEM((_RCH,), jnp.int32),
                       pltpu.VMEM((_RCH,), jnp.int32),
                       pltpu.VMEM((_RCH, _H), jnp.float32),
                       pltpu.VMEM((_RCH, _H), jnp.float32),
                       pltpu.SemaphoreType.DMA,
                       pltpu.SemaphoreType.DMA,
                       pltpu.SemaphoreType.DMA,
                       pltpu.SemaphoreType.DMA,
                       pltpu.SemaphoreType.DMA,
                       pltpu.SemaphoreType.DMA],
    )
    def k(table_hbm, idx_hbm, out_hbm, i0, i1, r0, r1,
          si0, si1, sg0, sg1, so0, so1):
        base = _wid() * _EPW
        iv, rv = [i0, i1], [r0, r1]
        si, sg, so = [si0, si1], [sg0, sg1], [so0, so1]
        icp = [None, None]
        ocp = [None, None]
        icp[0] = pltpu.async_copy(idx_hbm.at[pl.ds(base, _RCH)], iv[0], si[0])
        for c in range(_NRCH):
            p = c & 1
            if ocp[p] is not None:
                ocp[p].wait()
            icp[p].wait()
            g = pltpu.async_copy(table_hbm.at[iv[p]], rv[p], sg[p])
            if c + 1 < _NRCH:
                icp[1 - p] = pltpu.async_copy(
                    idx_hbm.at[pl.ds(base + (c + 1) * _RCH, _RCH)],
                    iv[1 - p], si[1 - p])
            g.wait()
            ocp[p] = pltpu.async_copy(
                rv[p], out_hbm.at[pl.ds(base + c * _RCH, _RCH)], so[p])
        ocp[0].wait()
        ocp[1].wait()

    return k(table, idx)


def kernel(batch, params):
    p = params
    f32 = jnp.float32
    w1 = jnp.zeros((8, _H), f32).at[:_DIN].set(p['enc_W1'])
    b1 = p['enc_b1'].reshape(1, _H)
    w2 = p['enc_W2']
    b2 = p['enc_b2'].reshape(1, _H)
    w3 = jnp.zeros((_H, 8), f32).at[:, :_DOUT].set(p['enc_W3'])
    b3 = jnp.zeros((1, 8), f32).at[0, :_DOUT].set(p['enc_b3'])
    ws = jnp.zeros((8, _H), f32).at[:_DOUT].set(p['skip_W'])
    bs = p['skip_b'].reshape(1, _H)
    lw = jnp.zeros((_H, 8), f32).at[:, :4].set(p['lab_W'])
    lb = jnp.zeros((1, 8), f32).at[0, :4].set(p['lab_b'])
    vw = jnp.zeros((_H, 8), f32).at[:, :1].set(p['val_W'])
    vb = jnp.zeros((1, 8), f32).at[0, :1].set(p['val_b'])

    labs, vals = [], []
    for b in range(_B):
        xp = jnp.zeros((_NPAD, 8), f32).at[:_N, :_DIN].set(batch[b])
        lat, skip = _encoder(xp, w1, b1, w2, b2, w3, b3, ws, bs)

        px = lat[:, 0]
        py = lat[:, 1]
        posT = jnp.zeros((8, _NPAD), f32).at[0].set(px).at[1].set(py)
        nbr = _knn(lat, posT, px.reshape(1, _NPAD), py.reshape(1, _NPAD))
        idx = nbr.reshape(-1)

        lat128 = jnp.zeros((_NPAD, _H), f32).at[:, :8].set(lat)
        latg = _sc_row_gather(lat128, idx)

        xl2, xr2, dist = _g1(latg, lat,
                             p['g1_Wl'], p['g1_Wr'], p['g1_We'],
                             p['g1_att'].reshape(1, _H),
                             p['g1_b'].reshape(1, _H),
                             p['g2_Wl'], p['g2_Wr'])

        xg2 = _sc_row_gather(xl2, idx)
        x2, xr3, xr4 = _g2(xg2, xr2, dist,
                           p['g2_We'], p['g2_att'].reshape(1, _H),
                           p['g2_b'].reshape(1, _H), p['g3_Wr'], p['g4_Wr'])

        xg = _sc_row_gather(x2, idx)
        lab, val = _g34(xg, xr3, xr4, skip,
                        p['g3_Wl'], p['g3_att'].reshape(1, _H),
                        p['g3_b'].reshape(1, _H),
                        p['g4_Wl'], p['g4_att'].reshape(1, _H),
                        p['g4_b'].reshape(1, _H),
                        lw, lb, vw, vb)
        labs.append(lab[:_N, :4])
        vals.append(val[:_N, :1])

    return jnp.stack(labs), jnp.stack(vals)
